# baseline (speedup 1.0000x reference)
import jax
import jax.numpy as jnp
from jax.experimental import pallas as pl

N = 10000
E = 160000
HEADS = 4
HID = 256


def _mm(x, W):
    M, K = x.shape
    _, Nc = W.shape
    BM = 1000

    def body(x_ref, w_ref, o_ref):
        o_ref[...] = jnp.dot(x_ref[...], w_ref[...],
                             preferred_element_type=jnp.float32)

    return pl.pallas_call(
        body,
        grid=(M // BM,),
        in_specs=[pl.BlockSpec((BM, K), lambda i: (i, 0)),
                  pl.BlockSpec((K, Nc), lambda i: (0, 0))],
        out_specs=pl.BlockSpec((BM, Nc), lambda i: (i, 0)),
        out_shape=jax.ShapeDtypeStruct((M, Nc), jnp.float32),
    )(x, W)


def _gat(x, src, dst, W, a_src, a_dst, bias, heads, out_ch):
    n = x.shape[0]
    h = _mm(x, W).reshape(n, heads, out_ch)
    al = (h * a_src).sum(-1)
    ar = (h * a_dst).sum(-1)
    e = al[src] + ar[dst]
    e = jax.nn.leaky_relu(e, 0.2)
    emax = jax.ops.segment_max(e, dst, num_segments=n)
    emax = jnp.where(jnp.isfinite(emax), emax, 0.0)
    ee = jnp.exp(e - emax[dst])
    denom = jax.ops.segment_sum(ee, dst, num_segments=n)
    alpha = ee / (denom[dst] + 1e-16)
    msg = h[src] * alpha[:, :, None]
    out = jax.ops.segment_sum(msg, dst, num_segments=n)
    return out.reshape(n, heads * out_ch) + bias


def _bn(x, g, b):
    m = x.mean(0)
    v = x.var(0)
    return (x - m) / jnp.sqrt(v + 1e-5) * g + b


def kernel(x, edge_index, W1, a_src1, a_dst1, b1, W2, a_src2, a_dst2, b2,
           W3, a_src3, a_dst3, b3, bn1_g, bn1_b, bn2_g, bn2_b,
           fc1_W, fc1_b, fc2_W, fc2_b):
    n = x.shape[0]
    loop = jnp.arange(n, dtype=edge_index.dtype)
    src = jnp.concatenate([edge_index[0], loop])
    dst = jnp.concatenate([edge_index[1], loop])
    h = _gat(x, src, dst, W1, a_src1, a_dst1, b1, HEADS, HID)
    h = jax.nn.elu(_bn(h, bn1_g, bn1_b))
    h = _gat(h, src, dst, W2, a_src2, a_dst2, b2, HEADS, HID)
    h = jax.nn.elu(_bn(h, bn2_g, bn2_b))
    h = _gat(h, src, dst, W3, a_src3, a_dst3, b3, 1, HID)
    h = jax.nn.elu(h)
    h = jax.nn.relu(_mm(h, fc1_W) + fc1_b)
    ncls = fc2_W.shape[1]
    fc2_pad = jnp.pad(fc2_W, ((0, 0), (0, 128 - ncls)))
    return _mm(h, fc2_pad)[:, :ncls] + fc2_b


# speedup vs baseline: 1.1318x; 1.1318x over previous
import functools

import jax
import jax.numpy as jnp
from jax import lax
from jax.experimental import pallas as pl
from jax.experimental.pallas import tpu as pltpu
from jax.experimental.pallas import tpu_sc as plsc

N = 10000
E = 160000
HEADS = 4
HID = 256

NC = 2
NS = 16
NW = NC * NS
CH = 64
E_LOOP = E + N
PER_W = 5376
E_PAD = PER_W * NW


def _mm(x, W):
    M, K = x.shape
    _, Nc = W.shape
    BM = 1000

    def body(x_ref, w_ref, o_ref):
        o_ref[...] = jnp.dot(x_ref[...], w_ref[...],
                             preferred_element_type=jnp.float32)

    return pl.pallas_call(
        body,
        grid=(M // BM,),
        in_specs=[pl.BlockSpec((BM, K), lambda i: (i, 0)),
                  pl.BlockSpec((K, Nc), lambda i: (0, 0))],
        out_specs=pl.BlockSpec((BM, Nc), lambda i: (i, 0)),
        out_shape=jax.ShapeDtypeStruct((M, Nc), jnp.float32),
    )(x, W)


def _make_gather(D):
    mesh = plsc.VectorSubcoreMesh(core_axis_name="c", subcore_axis_name="s")

    @functools.partial(
        pl.kernel, mesh=mesh,
        out_type=[
            jax.ShapeDtypeStruct((E_PAD, D), jnp.float32),
            jax.ShapeDtypeStruct((E_PAD, 128), jnp.float32),
            jax.ShapeDtypeStruct((E_PAD, 128), jnp.float32),
        ],
        scratch_types=[
            pltpu.VMEM((CH,), jnp.int32),
            pltpu.VMEM((CH,), jnp.int32),
            pltpu.VMEM((CH, D), jnp.float32),
            pltpu.VMEM((CH, 128), jnp.float32),
            pltpu.VMEM((CH, 128), jnp.float32),
            pltpu.SemaphoreType.DMA,
        ],
    )
    def k(h_hbm, tab_hbm, src_hbm, dst_hbm,
          hg_out, ag_out, bg_out, idx_s, idx_d, rows_v, a_v, b_v, sem):
        wid = lax.axis_index("s") * NC + lax.axis_index("c")
        base = wid * PER_W

        def chunk(c, _):
            off = base + c * CH
            pltpu.sync_copy(src_hbm.at[pl.ds(off, CH)], idx_s)
            pltpu.sync_copy(dst_hbm.at[pl.ds(off, CH)], idx_d)
            pltpu.async_copy(h_hbm.at[idx_s], rows_v, sem).wait()
            pltpu.sync_copy(rows_v, hg_out.at[pl.ds(off, CH)])
            pltpu.async_copy(tab_hbm.at[idx_s], a_v, sem).wait()
            pltpu.sync_copy(a_v, ag_out.at[pl.ds(off, CH)])
            pltpu.async_copy(tab_hbm.at[idx_d], b_v, sem).wait()
            pltpu.sync_copy(b_v, bg_out.at[pl.ds(off, CH)])
            return _

        lax.fori_loop(0, PER_W // CH, chunk, None)

    return k


_gather_1024 = _make_gather(1024)
_gather_256 = _make_gather(256)


def _gat(h, src_g, dst_g, dst_s, a_src, a_dst, bias, heads, out_ch, gather):
    n = h.shape[0]
    hh = h.reshape(n, heads, out_ch)
    al = (hh * a_src).sum(-1)
    ar = (hh * a_dst).sum(-1)
    M = jax.nn.leaky_relu(al.max(0) + ar.max(0), 0.2)
    tab = jnp.zeros((n, 128), jnp.float32)
    tab = tab.at[:, :heads].set(al).at[:, heads:2 * heads].set(ar)
    hg, ag, bg = gather(h, tab, src_g, dst_g)
    e = jax.nn.leaky_relu(ag[:, :heads] + bg[:, heads:2 * heads], 0.2)
    ee = jnp.exp(e - M[None, :])
    denom = jax.ops.segment_sum(ee, dst_s, num_segments=n)
    msg = hg.reshape(E_PAD, heads, out_ch) * ee[:, :, None]
    out = jax.ops.segment_sum(msg, dst_s, num_segments=n)
    out = out / (denom[:, :, None] + 1e-16)
    return out.reshape(n, heads * out_ch) + bias


def _bn(x, g, b):
    m = x.mean(0)
    v = x.var(0)
    return (x - m) / jnp.sqrt(v + 1e-5) * g + b


def kernel(x, edge_index, W1, a_src1, a_dst1, b1, W2, a_src2, a_dst2, b2,
           W3, a_src3, a_dst3, b3, bn1_g, bn1_b, bn2_g, bn2_b,
           fc1_W, fc1_b, fc2_W, fc2_b):
    n = x.shape[0]
    loop = jnp.arange(n, dtype=edge_index.dtype)
    pad = E_PAD - E_LOOP
    src = jnp.concatenate([edge_index[0], loop,
                           jnp.zeros((pad,), edge_index.dtype)])
    dst_g = jnp.concatenate([edge_index[1], loop,
                             jnp.zeros((pad,), edge_index.dtype)])
    dst_s = jnp.concatenate([edge_index[1], loop,
                             jnp.full((pad,), n, edge_index.dtype)])

    h = _mm(x, W1)
    h = _gat(h, src, dst_g, dst_s, a_src1, a_dst1, b1, HEADS, HID,
             _gather_1024)
    h = jax.nn.elu(_bn(h, bn1_g, bn1_b))
    h = _mm(h, W2)
    h = _gat(h, src, dst_g, dst_s, a_src2, a_dst2, b2, HEADS, HID,
             _gather_1024)
    h = jax.nn.elu(_bn(h, bn2_g, bn2_b))
    h = _mm(h, W3)
    h = _gat(h, src, dst_g, dst_s, a_src3, a_dst3, b3, 1, HID, _gather_256)
    h = jax.nn.elu(h)
    h = jax.nn.relu(_mm(h, fc1_W) + fc1_b)
    ncls = fc2_W.shape[1]
    fc2_pad = jnp.pad(fc2_W, ((0, 0), (0, 128 - ncls)))
    return _mm(h, fc2_pad)[:, :ncls] + fc2_b


# speedup vs baseline: 5.9591x; 5.2652x over previous
import functools

import jax
import jax.numpy as jnp
from jax import lax
from jax.experimental import pallas as pl
from jax.experimental.pallas import tpu as pltpu
from jax.experimental.pallas import tpu_sc as plsc

N = 10000
E = 160000
HEADS = 4
HID = 256

NC = 2
NS = 16
NW = NC * NS
CH = 64
E_LOOP = E + N
PER_W = 5376
E_PAD = PER_W * NW
PER_T = E_PAD // NS
STRIPE = 624


def _mm(x, W):
    M, K = x.shape
    _, Nc = W.shape
    BM = 1000

    def body(x_ref, w_ref, o_ref):
        o_ref[...] = jnp.dot(x_ref[...], w_ref[...],
                             preferred_element_type=jnp.float32)

    return pl.pallas_call(
        body,
        grid=(M // BM,),
        in_specs=[pl.BlockSpec((BM, K), lambda i: (i, 0)),
                  pl.BlockSpec((K, Nc), lambda i: (0, 0))],
        out_specs=pl.BlockSpec((BM, Nc), lambda i: (i, 0)),
        out_shape=jax.ShapeDtypeStruct((M, Nc), jnp.float32),
    )(x, W)


def _make_tab_gather():
    mesh = plsc.VectorSubcoreMesh(core_axis_name="c", subcore_axis_name="s")

    @functools.partial(
        pl.kernel, mesh=mesh,
        out_type=[
            jax.ShapeDtypeStruct((E_PAD, 128), jnp.float32),
            jax.ShapeDtypeStruct((E_PAD, 128), jnp.float32),
        ],
        scratch_types=[
            pltpu.VMEM((CH,), jnp.int32),
            pltpu.VMEM((CH,), jnp.int32),
            pltpu.VMEM((CH, 128), jnp.float32),
            pltpu.VMEM((CH, 128), jnp.float32),
            pltpu.SemaphoreType.DMA,
        ],
    )
    def k(tab_hbm, src_hbm, dst_hbm,
          ag_out, bg_out, idx_s, idx_d, a_v, b_v, sem):
        wid = lax.axis_index("s") * NC + lax.axis_index("c")
        base = wid * PER_W

        def chunk(c, _):
            off = base + c * CH
            pltpu.sync_copy(src_hbm.at[pl.ds(off, CH)], idx_s)
            pltpu.sync_copy(dst_hbm.at[pl.ds(off, CH)], idx_d)
            pltpu.async_copy(tab_hbm.at[idx_s], a_v, sem).wait()
            pltpu.sync_copy(a_v, ag_out.at[pl.ds(off, CH)])
            pltpu.async_copy(tab_hbm.at[idx_d], b_v, sem).wait()
            pltpu.sync_copy(b_v, bg_out.at[pl.ds(off, CH)])
            return _

        lax.fori_loop(0, PER_W // CH, chunk, None)

    return k


def _make_agg(nb, bph):
    mesh = plsc.VectorSubcoreMesh(core_axis_name="c", subcore_axis_name="s")

    @functools.partial(
        pl.kernel, mesh=mesh,
        out_type=jax.ShapeDtypeStruct((nb * N, 128), jnp.float32),
        scratch_types=[
            pltpu.VMEM_SHARED((N, 128), jnp.float32),
            pltpu.VMEM((CH,), jnp.int32),
            pltpu.VMEM((CH,), jnp.int32),
            pltpu.VMEM((CH,), jnp.float32),
            pltpu.VMEM((CH, 128), jnp.float32),
            pltpu.SemaphoreType.DMA,
        ],
    )
    def k(ht_hbm, eet_hbm, src_hbm, dst_hbm, zeros_hbm,
          out_hbm, acc, idx_s, idx_d, ee_v, rows_v, sem):
        sc = lax.axis_index("c")
        sid = lax.axis_index("s")

        def round_body(r, _):
            b = r * NC + sc
            hd = b // bph

            pltpu.sync_copy(zeros_hbm.at[pl.ds(sid * STRIPE, STRIPE)],
                            acc.at[pl.ds(sid * STRIPE, STRIPE)])

            @pl.when(sid == 0)
            def _tail_zero():
                pltpu.sync_copy(zeros_hbm.at[pl.ds(NS * STRIPE,
                                                   N - NS * STRIPE)],
                                acc.at[pl.ds(NS * STRIPE, N - NS * STRIPE)])

            plsc.subcore_barrier()

            def chunk(c, _):
                off = sid * PER_T + c * CH
                pltpu.sync_copy(src_hbm.at[pl.ds(off, CH)], idx_s)
                pltpu.sync_copy(dst_hbm.at[pl.ds(off, CH)], idx_d)
                pltpu.sync_copy(eet_hbm.at[pl.ds(hd * E_PAD + off, CH)],
                                ee_v)
                shift = jnp.full((16,), b * N, jnp.int32)
                for g in range(CH // 16):
                    sl = pl.ds(g * 16, 16)
                    idx_s[sl] = idx_s[sl] + shift
                pltpu.async_copy(ht_hbm.at[idx_s], rows_v, sem).wait()

                for g in range(CH // 16):
                    gv = ee_v[pl.ds(g * 16, 16)]
                    for t in range(16):
                        sv = jnp.full((16,), gv[t], jnp.float32)
                        row = g * 16 + t
                        for j in range(8):
                            sl = pl.ds(j * 16, 16)
                            rows_v[row, sl] = rows_v[row, sl] * sv

                pltpu.sync_copy(rows_v, acc.at[idx_d], add=True)
                return _

            lax.fori_loop(0, PER_T // CH, chunk, None)
            plsc.subcore_barrier()

            pltpu.sync_copy(acc.at[pl.ds(sid * STRIPE, STRIPE)],
                            out_hbm.at[pl.ds(b * N + sid * STRIPE, STRIPE)])

            @pl.when(sid == 0)
            def _tail_out():
                pltpu.sync_copy(acc.at[pl.ds(NS * STRIPE, N - NS * STRIPE)],
                                out_hbm.at[pl.ds(b * N + NS * STRIPE,
                                                 N - NS * STRIPE)])

            plsc.subcore_barrier()
            return _

        lax.fori_loop(0, nb // NC, round_body, None)

    return k


_tab_gather = _make_tab_gather()
_agg_1024 = _make_agg(8, 2)
_agg_256 = _make_agg(2, 2)


def _gat(h, src_g, dst_g, dst_s, valid, a_src, a_dst, bias, heads, out_ch,
         agg, zeros):
    n = h.shape[0]
    nb = (heads * out_ch) // 128
    hh = h.reshape(n, heads, out_ch)
    al = (hh * a_src).sum(-1)
    ar = (hh * a_dst).sum(-1)
    M = jax.nn.leaky_relu(al.max(0) + ar.max(0), 0.2)
    tab = jnp.zeros((n, 128), jnp.float32)
    tab = tab.at[:, :heads].set(al).at[:, heads:2 * heads].set(ar)
    ag, bg = _tab_gather(tab, src_g, dst_g)
    e = jax.nn.leaky_relu(ag[:, :heads] + bg[:, heads:2 * heads], 0.2)
    ee = jnp.exp(e - M[None, :]) * valid[:, None]
    denom = jax.ops.segment_sum(ee, dst_s, num_segments=n)
    h_t = h.reshape(n, nb, 128).swapaxes(0, 1).reshape(nb * n, 128)
    eet = ee.T.reshape(heads * E_PAD)
    out_t = agg(h_t, eet, src_g, dst_g, zeros)
    out = out_t.reshape(nb, n, 128).swapaxes(0, 1).reshape(n, heads, out_ch)
    out = out / (denom[:, :, None] + 1e-16)
    return out.reshape(n, heads * out_ch) + bias


def _bn(x, g, b):
    m = x.mean(0)
    v = x.var(0)
    return (x - m) / jnp.sqrt(v + 1e-5) * g + b


def kernel(x, edge_index, W1, a_src1, a_dst1, b1, W2, a_src2, a_dst2, b2,
           W3, a_src3, a_dst3, b3, bn1_g, bn1_b, bn2_g, bn2_b,
           fc1_W, fc1_b, fc2_W, fc2_b):
    n = x.shape[0]
    loop = jnp.arange(n, dtype=edge_index.dtype)
    pad = E_PAD - E_LOOP
    src = jnp.concatenate([edge_index[0], loop,
                           jnp.zeros((pad,), edge_index.dtype)])
    dst_g = jnp.concatenate([edge_index[1], loop,
                             jnp.zeros((pad,), edge_index.dtype)])
    dst_s = jnp.concatenate([edge_index[1], loop,
                             jnp.full((pad,), n, edge_index.dtype)])
    valid = (jnp.arange(E_PAD) < E_LOOP).astype(jnp.float32)
    zeros = jnp.zeros((N, 128), jnp.float32)

    h = _mm(x, W1)
    h = _gat(h, src, dst_g, dst_s, valid, a_src1, a_dst1, b1, HEADS, HID,
             _agg_1024, zeros)
    h = jax.nn.elu(_bn(h, bn1_g, bn1_b))
    h = _mm(h, W2)
    h = _gat(h, src, dst_g, dst_s, valid, a_src2, a_dst2, b2, HEADS, HID,
             _agg_1024, zeros)
    h = jax.nn.elu(_bn(h, bn2_g, bn2_b))
    h = _mm(h, W3)
    h = _gat(h, src, dst_g, dst_s, valid, a_src3, a_dst3, b3, 1, HID,
             _agg_256, zeros)
    h = jax.nn.elu(h)
    h = jax.nn.relu(_mm(h, fc1_W) + fc1_b)
    ncls = fc2_W.shape[1]
    fc2_pad = jnp.pad(fc2_W, ((0, 0), (0, 128 - ncls)))
    return _mm(h, fc2_pad)[:, :ncls] + fc2_b
